# baseline (speedup 1.0000x reference)
import functools

import jax
import jax.numpy as jnp
from jax import lax
from jax.experimental import pallas as pl
from jax.experimental.pallas import tpu as pltpu
from jax.experimental.pallas import tpu_sc as plsc

_B = 16384
_D = 128
_NC = 2
_NS = 16
_NW = _NC * _NS
_BPW = _B // _NW
_NCH = 16
_CH = _BPW // _NCH


def _emotion_lookup_sc(table_hbm, idx_hbm, out_hbm, idx_v, rows_v, tbl_sh,
                       gsem, osem):
    sid = lax.axis_index("s")
    wid = sid * _NC + lax.axis_index("c")
    base = wid * _BPW

    @pl.when(sid == 0)
    def _():
        pltpu.sync_copy(table_hbm, tbl_sh)

    pltpu.sync_copy(idx_hbm.at[pl.ds(base, _BPW)], idx_v)
    plsc.subcore_barrier()

    gathers = []
    for k in range(_NCH):
        gathers.append(
            pltpu.async_copy(tbl_sh.at[idx_v.at[pl.ds(k * _CH, _CH)]],
                             rows_v.at[k], gsem))
    outs = []
    for k in range(_NCH):
        gathers[k].wait()
        outs.append(
            pltpu.async_copy(rows_v.at[k],
                             out_hbm.at[pl.ds(base + k * _CH, _CH)], osem))
    for k in range(_NCH):
        outs[k].wait()


@jax.jit
def kernel(emotion_id, embedding_weight):
    mesh = plsc.VectorSubcoreMesh(core_axis_name="c", subcore_axis_name="s")
    run = functools.partial(
        pl.kernel,
        mesh=mesh,
        out_type=jax.ShapeDtypeStruct((_B, _D), jnp.float32),
        scratch_types=[
            pltpu.VMEM((_BPW,), jnp.int32),
            pltpu.VMEM((_NCH, _CH, _D), jnp.float32),
            pltpu.VMEM_SHARED((8, _D), jnp.float32),
            pltpu.SemaphoreType.DMA,
            pltpu.SemaphoreType.DMA,
        ],
    )(_emotion_lookup_sc)
    return run(embedding_weight, emotion_id.astype(jnp.int32))


# speedup vs baseline: 1.0140x; 1.0140x over previous
import functools

import jax
import jax.numpy as jnp
from jax import lax
from jax.experimental import pallas as pl
from jax.experimental.pallas import tpu as pltpu
from jax.experimental.pallas import tpu_sc as plsc

_B = 16384
_D = 128
_E = 8
_NC = 2
_NS = 16
_NW = _NC * _NS
_BPW = _B // _NW
_NCH = 8
_CH = _BPW // _NCH


def _emotion_lookup_sc(table_hbm, idx_hbm, out_hbm, idx_v, rows_v, tbl_sh,
                       gsem, osem):
    sid = lax.axis_index("s")
    wid = sid * _NC + lax.axis_index("c")
    base = wid * _BPW

    @pl.when(sid == 0)
    def _():
        pltpu.sync_copy(table_hbm, tbl_sh)

    pltpu.sync_copy(idx_hbm.at[pl.ds(base, _BPW)], idx_v)
    plsc.subcore_barrier()

    gathers = []
    for k in range(_NCH):
        gathers.append(
            pltpu.async_copy(tbl_sh.at[idx_v.at[pl.ds(k * _CH, _CH)]],
                             rows_v.at[k], gsem))
    outs = []
    for k in range(_NCH):
        gathers[k].wait()
        outs.append(
            pltpu.async_copy(rows_v.at[k],
                             out_hbm.at[pl.ds(base + k * _CH, _CH)], osem))
    for k in range(_NCH):
        outs[k].wait()


@jax.jit
def kernel(emotion_id, embedding_weight):
    mesh = plsc.VectorSubcoreMesh(core_axis_name="c", subcore_axis_name="s")
    run = functools.partial(
        pl.kernel,
        mesh=mesh,
        out_type=jax.ShapeDtypeStruct((_B, _D), jnp.float32),
        scratch_types=[
            pltpu.VMEM((_BPW,), jnp.int32),
            pltpu.VMEM((_NCH, _CH, _D), jnp.float32),
            pltpu.VMEM_SHARED((_E, _D), jnp.float32),
            pltpu.SemaphoreType.DMA,
            pltpu.SemaphoreType.DMA,
        ],
    )(_emotion_lookup_sc)
    return run(embedding_weight, emotion_id.astype(jnp.int32))


# speedup vs baseline: 1.0378x; 1.0234x over previous
import functools

import jax
import jax.numpy as jnp
from jax import lax
from jax.experimental import pallas as pl
from jax.experimental.pallas import tpu as pltpu
from jax.experimental.pallas import tpu_sc as plsc

_B = 16384
_D = 128
_E = 8
_NC = 2
_NS = 16
_NW = _NC * _NS
_BPW = _B // _NW
_NCH = 8
_CH = _BPW // _NCH


def _emotion_lookup_sc(table_hbm, idx_hbm, out_hbm, idx_v, rows_v, tbl_sh,
                       gsem, osem):
    sid = lax.axis_index("s")
    wid = sid * _NC + lax.axis_index("c")
    base = wid * _BPW

    icp = pltpu.async_copy(idx_hbm.at[pl.ds(base, _BPW)], idx_v, osem)

    @pl.when(sid == 0)
    def _():
        pltpu.sync_copy(table_hbm, tbl_sh)

    icp.wait()
    plsc.subcore_barrier()

    gathers = []
    for k in range(_NCH):
        gathers.append(
            pltpu.async_copy(tbl_sh.at[idx_v.at[pl.ds(k * _CH, _CH)]],
                             rows_v.at[k], gsem))
    outs = []
    for k in range(_NCH):
        gathers[k].wait()
        outs.append(
            pltpu.async_copy(rows_v.at[k],
                             out_hbm.at[pl.ds(base + k * _CH, _CH)], osem))
    for k in range(_NCH):
        outs[k].wait()


@jax.jit
def kernel(emotion_id, embedding_weight):
    mesh = plsc.VectorSubcoreMesh(core_axis_name="c", subcore_axis_name="s")
    run = functools.partial(
        pl.kernel,
        mesh=mesh,
        out_type=jax.ShapeDtypeStruct((_B, _D), jnp.float32),
        scratch_types=[
            pltpu.VMEM((_BPW,), jnp.int32),
            pltpu.VMEM((_NCH, _CH, _D), jnp.float32),
            pltpu.VMEM_SHARED((_E, _D), jnp.float32),
            pltpu.SemaphoreType.DMA,
            pltpu.SemaphoreType.DMA,
        ],
    )(_emotion_lookup_sc)
    return run(embedding_weight, emotion_id.astype(jnp.int32))


# speedup vs baseline: 1.0482x; 1.0101x over previous
import functools

import jax
import jax.numpy as jnp
from jax import lax
from jax.experimental import pallas as pl
from jax.experimental.pallas import tpu as pltpu
from jax.experimental.pallas import tpu_sc as plsc

_B = 16384
_D = 128
_E = 8
_NC = 2
_NS = 16
_NW = _NC * _NS
_BPW = _B // _NW
_NCH = 4
_CH = _BPW // _NCH


def _emotion_lookup_sc(table_hbm, idx_hbm, out_hbm, idx_v, rows_v, tbl_sh,
                       gsem, osem):
    sid = lax.axis_index("s")
    wid = sid * _NC + lax.axis_index("c")
    base = wid * _BPW

    icp = pltpu.async_copy(idx_hbm.at[pl.ds(base, _BPW)], idx_v, osem)

    @pl.when(sid == 0)
    def _():
        pltpu.sync_copy(table_hbm, tbl_sh)

    icp.wait()
    plsc.subcore_barrier()

    gathers = []
    for k in range(_NCH):
        gathers.append(
            pltpu.async_copy(tbl_sh.at[idx_v.at[pl.ds(k * _CH, _CH)]],
                             rows_v.at[k], gsem))
    outs = []
    for k in range(_NCH):
        gathers[k].wait()
        outs.append(
            pltpu.async_copy(rows_v.at[k],
                             out_hbm.at[pl.ds(base + k * _CH, _CH)], osem))
    for k in range(_NCH):
        outs[k].wait()


@jax.jit
def kernel(emotion_id, embedding_weight):
    mesh = plsc.VectorSubcoreMesh(core_axis_name="c", subcore_axis_name="s")
    run = functools.partial(
        pl.kernel,
        mesh=mesh,
        out_type=jax.ShapeDtypeStruct((_B, _D), jnp.float32),
        scratch_types=[
            pltpu.VMEM((_BPW,), jnp.int32),
            pltpu.VMEM((_NCH, _CH, _D), jnp.float32),
            pltpu.VMEM_SHARED((_E, _D), jnp.float32),
            pltpu.SemaphoreType.DMA,
            pltpu.SemaphoreType.DMA,
        ],
    )(_emotion_lookup_sc)
    return run(embedding_weight, emotion_id.astype(jnp.int32))
